# baseline (speedup 1.0000x reference)
import jax
import jax.numpy as jnp
from jax import lax
from jax.experimental import pallas as pl
from jax.experimental.pallas import tpu as pltpu
from jax.experimental.pallas import tpu_sc as plsc

_N = 10000
_E = 320000
_D = 128
_NC = 2
_NS = 16
_NW = _NC * _NS
_EPW = _E // _NW
_K = 80
_NCHUNK = _EPW // _K
_RPT0 = 632
_RPTL = _N - (_NS - 1) * _RPT0


def _linear_body(x_ref, w_ref, b_ref, o_ref):
    o_ref[...] = lax.dot_general(
        x_ref[...], w_ref[...], (((1,), (1,)), ((), ())),
        preferred_element_type=jnp.float32) + b_ref[...]


def _linear(x, W, b):
    bm = 1000
    return pl.pallas_call(
        _linear_body,
        grid=(_N // bm,),
        in_specs=[
            pl.BlockSpec((bm, _D), lambda i: (i, 0)),
            pl.BlockSpec((_D, _D), lambda i: (0, 0)),
            pl.BlockSpec((1, _D), lambda i: (0, 0)),
        ],
        out_specs=pl.BlockSpec((bm, _D), lambda i: (i, 0)),
        out_shape=jax.ShapeDtypeStruct((_N, _D), jnp.float32),
    )(x, W, b.reshape(1, _D))


def _splat_lane(vec16, lane):
    return lax.gather(
        vec16, jnp.full((16, 1), lane, jnp.int32),
        lax.GatherDimensionNumbers(
            offset_dims=(), collapsed_slice_dims=(0,), start_index_map=(0,)),
        slice_sizes=(1,),
        mode=lax.GatherScatterMode.PROMISE_IN_BOUNDS)


def _sc_agg_body(y_hbm, pk_hbm, out_hbm,
                 pbuf, rbuf, gbuf, acc,
                 gsem0, gsem1, gsem2, psem0, psem1, psem2,
                 ssem0, ssem1, ssem2):
    c = lax.axis_index("c")
    s = lax.axis_index("s")
    wid = s * _NC + c
    gsems = (gsem0, gsem1, gsem2)
    psems = (psem0, psem1, psem2)
    ssems = (ssem0, ssem1, ssem2)

    def _zrow(r, carry):
        for j in range(_D // 16):
            gbuf[0, r, pl.ds(j * 16, 16)] = jnp.zeros((16,), jnp.float32)
        return carry
    lax.fori_loop(0, _K, _zrow, 0)

    @pl.when(s < _NS - 1)
    def _():
        for q in range(_RPT0 // _K):
            pltpu.sync_copy(gbuf.at[0],
                            acc.at[pl.ds(s * _RPT0 + q * _K, _K)])
        rem = _RPT0 % _K
        pltpu.sync_copy(gbuf.at[0, pl.ds(0, rem)],
                        acc.at[pl.ds(s * _RPT0 + _RPT0 - rem, rem)])

    @pl.when(s == _NS - 1)
    def _():
        for q in range(_RPTL // _K):
            pltpu.sync_copy(gbuf.at[0],
                            acc.at[pl.ds(s * _RPT0 + q * _K, _K)])
        rem = _RPTL % _K
        pltpu.sync_copy(gbuf.at[0, pl.ds(0, rem)],
                        acc.at[pl.ds(s * _RPT0 + _RPTL - rem, rem)])

    pltpu.async_copy(pk_hbm.at[wid, 0], pbuf.at[0], psem0)
    pltpu.async_copy(pk_hbm.at[wid, 1], pbuf.at[1], psem1)
    pltpu.async_copy(pk_hbm.at[wid, 2], pbuf.at[2], psem2)
    plsc.subcore_barrier()
    pltpu.make_async_copy(pk_hbm.at[wid, 0], pbuf.at[0], psem0).wait()
    pltpu.async_copy(y_hbm.at[pbuf.at[0, 0]], gbuf.at[0], gsem0)
    pltpu.make_async_copy(pk_hbm.at[wid, 1], pbuf.at[1], psem1).wait()
    pltpu.async_copy(y_hbm.at[pbuf.at[1, 0]], gbuf.at[1], gsem1)

    def _multiply(b):
        def _grp(g, carry):
            sl16 = pl.ds(g * 16, 16)
            rbuf[b, sl16] = pbuf[b, 1, sl16]
            val16 = lax.bitcast_convert_type(pbuf[b, 2, sl16], jnp.float32)
            for e in range(16):
                vsplat = _splat_lane(val16, e)
                row = g * 16 + e
                for j in range(_D // 16):
                    slj = pl.ds(j * 16, 16)
                    gbuf[b, row, slj] = gbuf[b, row, slj] * vsplat
            return carry
        lax.fori_loop(0, _K // 16, _grp, 0)

    def _chunk(ci, b, b2, k=None, depth2=True):
        pltpu.make_async_copy(y_hbm.at[pbuf.at[b, 0]], gbuf.at[b],
                              gsems[b]).wait()
        _multiply(b)
        pltpu.async_copy(gbuf.at[b], acc.at[rbuf.at[b]], ssems[b], add=True)
        if depth2:
            pltpu.async_copy(
                pk_hbm.at[wid, jnp.minimum(ci + 3, _NCHUNK - 1)],
                pbuf.at[b], psems[b])
        def _wait_s():
            pltpu.make_async_copy(gbuf.at[b2], acc.at[rbuf.at[b2]],
                                  ssems[b2]).wait()
        if k is None:
            _wait_s()
        else:
            pl.when(k >= 1)(_wait_s)
        if depth2:
            pltpu.make_async_copy(pk_hbm.at[wid, ci + 2], pbuf.at[b2],
                                  psems[b2]).wait()
            pltpu.async_copy(y_hbm.at[pbuf.at[b2, 0]], gbuf.at[b2],
                             gsems[b2])

    def _kbody(k, carry):
        base = 3 * k
        _chunk(base + 0, 0, 2, k=k)
        _chunk(base + 1, 1, 0)
        _chunk(base + 2, 2, 1)
        return carry
    lax.fori_loop(0, 41, _kbody, 0)

    _chunk(_NCHUNK - 2, 0, 2, depth2=False)
    _chunk(_NCHUNK - 1, 1, 0, depth2=False)
    pltpu.make_async_copy(gbuf.at[1], acc.at[rbuf.at[1]], ssems[1]).wait()
    pltpu.make_async_copy(pk_hbm.at[wid, 0], pbuf.at[2], psems[2]).wait()

    plsc.subcore_barrier()
    base = c * _N + s * _RPT0

    @pl.when(s < _NS - 1)
    def _():
        pltpu.sync_copy(acc.at[pl.ds(s * _RPT0, _RPT0)],
                        out_hbm.at[pl.ds(base, _RPT0)])

    @pl.when(s == _NS - 1)
    def _():
        pltpu.sync_copy(acc.at[pl.ds(s * _RPT0, _RPTL)],
                        out_hbm.at[pl.ds(base, _RPTL)])


def _sc_agg(y, pk):
    mesh = plsc.VectorSubcoreMesh(core_axis_name="c", subcore_axis_name="s")
    fn = pl.kernel(
        _sc_agg_body,
        mesh=mesh,
        out_type=jax.ShapeDtypeStruct((_NC * _N, _D), jnp.float32),
        scratch_types=[
            pltpu.VMEM((3, 3, _K), jnp.int32),
            pltpu.VMEM((3, _K), jnp.int32),
            pltpu.VMEM((3, _K, _D), jnp.float32),
            pltpu.VMEM_SHARED((_N, _D), jnp.float32),
            pltpu.SemaphoreType.DMA,
            pltpu.SemaphoreType.DMA,
            pltpu.SemaphoreType.DMA,
            pltpu.SemaphoreType.DMA,
            pltpu.SemaphoreType.DMA,
            pltpu.SemaphoreType.DMA,
            pltpu.SemaphoreType.DMA,
            pltpu.SemaphoreType.DMA,
            pltpu.SemaphoreType.DMA,
        ],
    )
    return fn(y, pk)


def _combine_body(a_ref, b_ref, o_ref):
    o_ref[...] = a_ref[...] + b_ref[...]


def _combine(partials):
    bm = 1000
    nb = _N // bm
    return pl.pallas_call(
        _combine_body,
        grid=(nb,),
        in_specs=[
            pl.BlockSpec((bm, _D), lambda i: (i, 0)),
            pl.BlockSpec((bm, _D), lambda i: (i + nb, 0)),
        ],
        out_specs=pl.BlockSpec((bm, _D), lambda i: (i, 0)),
        out_shape=jax.ShapeDtypeStruct((_N, _D), jnp.float32),
    )(partials, partials)


def kernel(x, adj_indices, adj_values, W, b):
    row = adj_indices[0].astype(jnp.int32).reshape(_NW, _NCHUNK, _K)
    col = adj_indices[1].astype(jnp.int32).reshape(_NW, _NCHUNK, _K)
    valbits = lax.bitcast_convert_type(
        adj_values.astype(jnp.float32), jnp.int32).reshape(_NW, _NCHUNK, _K)
    pk = jnp.stack([col, row, valbits], axis=2)
    y = _linear(x, W, b)
    partials = _sc_agg(y, pk)
    return _combine(partials)
